# baseline (speedup 1.0000x reference)
import functools
import math

import jax
import jax.numpy as jnp
from jax import lax
from jax.experimental import pallas as pl
from jax.experimental.pallas import tpu as pltpu
from jax.experimental.pallas import tpu_sc as plsc

N = 10000
E = 320000
G = 64
D = 128
NUM_CLASSES = 16
EPS = 1e-5
SCALE = 1.0 / math.sqrt(1.0 + EPS)

NP = 10240
BN = 2560
NGRID = NP // BN

EC = 128
NCH = 2560
NT = 16
NW = 2 * NT
CW = NCH // NW
RPT = NP // NT
HC = 40

_f32 = jnp.float32


@functools.cache
def _mesh():
    return plsc.VectorSubcoreMesh(core_axis_name="c", subcore_axis_name="s",
                                  num_cores=2, num_subcores=NT)


def _deg_body(dstp, zeros_d, ones_e, dega, degb, di_v, ones_v, deg_sh):
    c = lax.axis_index("c")
    s = lax.axis_index("s")
    w = c * NT + s
    pltpu.sync_copy(zeros_d, deg_sh.at[pl.ds(s * RPT, RPT)])
    pltpu.sync_copy(ones_e, ones_v)
    pltpu.sync_copy(dstp.at[pl.ds(w * CW, CW)], di_v)
    plsc.subcore_barrier()

    def body(j, carry):
        pltpu.sync_copy(ones_v, deg_sh.at[di_v.at[j]], add=True)
        return carry

    lax.fori_loop(0, CW, body, 0)
    plsc.subcore_barrier()

    @pl.when(c == 0)
    def _():
        pltpu.sync_copy(deg_sh.at[pl.ds(s * RPT, RPT)],
                        dega.at[pl.ds(s * RPT, RPT)])

    @pl.when(c == 1)
    def _():
        pltpu.sync_copy(deg_sh.at[pl.ds(s * RPT, RPT)],
                        degb.at[pl.ds(s * RPT, RPT)])


@functools.cache
def _deg_call():
    return pl.kernel(
        _deg_body,
        out_type=(jax.ShapeDtypeStruct((NP,), _f32),
                  jax.ShapeDtypeStruct((NP,), _f32)),
        mesh=_mesh(),
        scratch_types=[
            pltpu.VMEM((CW, EC), jnp.int32),
            pltpu.VMEM((EC,), _f32),
            pltpu.VMEM_SHARED((NP,), _f32),
        ],
    )


def _scat_body(z, srcp, dstp, ya, yb, si_v, di_v,
               rows_a, rows_b, gsa, gsb, y_sh):
    c = lax.axis_index("c")
    s = lax.axis_index("s")
    w = c * NT + s

    zv = jnp.zeros((16,), _f32)

    def zbody(i, carry):
        for kk in range(8):
            rows_a.at[i][pl.ds(kk * 16, 16)] = zv
        return carry

    lax.fori_loop(0, EC, zbody, 0)
    for r in range(RPT // EC):
        pltpu.sync_copy(rows_a, y_sh.at[pl.ds(s * RPT + r * EC, EC)])
    plsc.subcore_barrier()

    for h in range(CW // HC):
        base = w * CW + h * HC
        pltpu.sync_copy(srcp.at[pl.ds(base, HC)], si_v)
        pltpu.sync_copy(dstp.at[pl.ds(base, HC)], di_v)
        pltpu.async_copy(z.at[si_v.at[0]], rows_a, gsa)
        pltpu.async_copy(z.at[si_v.at[1]], rows_b, gsb)

        def body(k, carry):
            j0 = 2 * k
            pltpu.make_async_copy(z.at[si_v.at[0]], rows_a, gsa).wait()
            pltpu.sync_copy(rows_a, y_sh.at[di_v.at[j0]], add=True)
            pltpu.async_copy(z.at[si_v.at[j0 + 2]], rows_a, gsa)
            pltpu.make_async_copy(z.at[si_v.at[0]], rows_b, gsb).wait()
            pltpu.sync_copy(rows_b, y_sh.at[di_v.at[j0 + 1]], add=True)
            pltpu.async_copy(z.at[si_v.at[j0 + 3]], rows_b, gsb)
            return carry

        lax.fori_loop(0, HC // 2 - 1, body, 0)
        pltpu.make_async_copy(z.at[si_v.at[0]], rows_a, gsa).wait()
        pltpu.sync_copy(rows_a, y_sh.at[di_v.at[HC - 2]], add=True)
        pltpu.make_async_copy(z.at[si_v.at[0]], rows_b, gsb).wait()
        pltpu.sync_copy(rows_b, y_sh.at[di_v.at[HC - 1]], add=True)

    plsc.subcore_barrier()

    @pl.when(c == 0)
    def _():
        pltpu.sync_copy(y_sh.at[pl.ds(s * RPT, RPT)],
                        ya.at[pl.ds(s * RPT, RPT)])

    @pl.when(c == 1)
    def _():
        pltpu.sync_copy(y_sh.at[pl.ds(s * RPT, RPT)],
                        yb.at[pl.ds(s * RPT, RPT)])


@functools.cache
def _scat_call():
    return pl.kernel(
        _scat_body,
        out_type=(jax.ShapeDtypeStruct((NP, D), _f32),
                  jax.ShapeDtypeStruct((NP, D), _f32)),
        mesh=_mesh(),
        scratch_types=[
            pltpu.VMEM((HC, EC), jnp.int32),
            pltpu.VMEM((HC, EC), jnp.int32),
            pltpu.VMEM((EC, D), _f32),
            pltpu.VMEM((EC, D), _f32),
            pltpu.SemaphoreType.DMA,
            pltpu.SemaphoreType.DMA,
            pltpu.VMEM_SHARED((NP, D), _f32),
        ],
    )


def _dis(dega, degb):
    return lax.rsqrt(1.0 + dega[...] + degb[...])


def _mm_body(x_ref, w_ref, u_ref):
    u_ref[...] = jnp.dot(x_ref[...], w_ref[...], preferred_element_type=_f32)


_row_spec = pl.BlockSpec((BN, D), lambda i: (i, 0))
_deg_spec = pl.BlockSpec((BN, 1), lambda i: (i, 0))
_vec_spec = pl.BlockSpec((1, D), lambda i: (0, 0))
_w_spec = pl.BlockSpec((D, D), lambda i: (0, 0))

_mm_call = pl.pallas_call(
    _mm_body,
    grid=(NGRID,),
    in_specs=[_row_spec, _w_spec],
    out_specs=_row_spec,
    out_shape=jax.ShapeDtypeStruct((NP, D), _f32),
)


def _scale_body(u_ref, dega, degb, z_ref):
    z_ref[...] = _dis(dega, degb) * u_ref[...]


_scale_call = pl.pallas_call(
    _scale_body,
    grid=(NGRID,),
    in_specs=[_row_spec, _deg_spec, _deg_spec],
    out_specs=_row_spec,
    out_shape=jax.ShapeDtypeStruct((NP, D), _f32),
)


def _h_block(ya, yb, z, dega, degb, bp, g, be):
    dis = _dis(dega, degb)
    h = (dis * (ya[...] + yb[...] + z[...]) + bp[...]) * (g[...] * SCALE)
    return dis, jnp.maximum(h + be[...], 0.0)


def _mid_body(ya, yb, z, dega, degb, bp, g, be, w_ref, zo):
    dis, h = _h_block(ya, yb, z, dega, degb, bp, g, be)
    zo[...] = dis * jnp.dot(h, w_ref[...], preferred_element_type=_f32)


_mid_call = pl.pallas_call(
    _mid_body,
    grid=(NGRID,),
    in_specs=[_row_spec] * 3 + [_deg_spec] * 2 + [_vec_spec] * 3 + [_w_spec],
    out_specs=_row_spec,
    out_shape=jax.ShapeDtypeStruct((NP, D), _f32),
)


def _pool_body(ya, yb, z, dega, degb, bp, g, be, bt,
               sc, sw, sb, w1g, w1s, b1, w2, b2,
               out_ref, p_acc, cnt_acc):
    i = pl.program_id(0)
    _, h = _h_block(ya, yb, z, dega, degb, bp, g, be)
    gid = lax.broadcasted_iota(jnp.int32, (1, G), 1)
    mask = (bt[...] == gid).astype(_f32)
    dn = (((0,), (0,)), ((), ()))
    p = lax.dot_general(mask, h, dn, preferred_element_type=_f32)
    cnt = lax.dot_general(mask, jnp.ones((BN, 1), _f32), dn,
                          preferred_element_type=_f32)

    @pl.when(i == 0)
    def _():
        p_acc[...] = jnp.zeros_like(p_acc)
        cnt_acc[...] = jnp.zeros_like(cnt_acc)

    p_acc[...] += p
    cnt_acc[...] += cnt

    @pl.when(i == NGRID - 1)
    def _():
        ge = p_acc[...] / jnp.maximum(cnt_acc[...], 1.0)
        se = jnp.maximum(jnp.dot(sc[...], sw[...],
                                 preferred_element_type=_f32) + sb[...], 0.0)
        hc = jnp.maximum(
            jnp.dot(ge, w1g[...], preferred_element_type=_f32)
            + jnp.dot(se, w1s[...], preferred_element_type=_f32)
            + b1[...], 0.0)
        out_ref[...] = jnp.dot(hc, w2[...],
                               preferred_element_type=_f32) + b2[...]


def _const_spec(r, c):
    return pl.BlockSpec((r, c), lambda i: (0, 0))


_pool_call = pl.pallas_call(
    _pool_body,
    grid=(NGRID,),
    in_specs=[_row_spec] * 3 + [_deg_spec] * 2 + [_vec_spec] * 3
             + [pl.BlockSpec((BN, 1), lambda i: (i, 0)),
                _const_spec(G, 8), _const_spec(8, D // 2),
                _const_spec(1, D // 2), _const_spec(D, D),
                _const_spec(D // 2, D), _const_spec(1, D),
                _const_spec(D, NUM_CLASSES), _const_spec(1, NUM_CLASSES)],
    out_specs=_const_spec(G, NUM_CLASSES),
    out_shape=jax.ShapeDtypeStruct((G, NUM_CLASSES), _f32),
    scratch_shapes=[pltpu.VMEM((G, D), _f32), pltpu.VMEM((G, 1), _f32)],
)


def kernel(x, edge_index, batch, scalar,
           conv_W0, conv_b0, conv_W1, conv_b1, conv_W2, conv_b2,
           bn_g0, bn_b0, bn_g1, bn_b1, bn_g2, bn_b2,
           scalar_W, scalar_b, cls_W1, cls_b1, cls_W2, cls_b2):
    npad = NCH * EC - E
    pads = N + (jnp.arange(npad, dtype=jnp.int32) % (NP - N))
    srcp = jnp.concatenate([edge_index[0], pads]).reshape(NCH, EC)
    dstp = jnp.concatenate([edge_index[1], pads]).reshape(NCH, EC)

    xp = jnp.pad(x, ((0, NP - N), (0, 0)))
    bt = jnp.concatenate(
        [batch, jnp.full((NP - N,), G, jnp.int32)]).reshape(NP, 1)

    zeros_d = jnp.zeros((RPT,), _f32)
    ones_e = jnp.ones((EC,), _f32)

    def vec(v):
        return v.reshape(1, v.shape[0])

    layers = ((conv_W0, vec(conv_b0), vec(bn_g0), vec(bn_b0)),
              (conv_W1, vec(conv_b1), vec(bn_g1), vec(bn_b1)),
              (conv_W2, vec(conv_b2), vec(bn_g2), vec(bn_b2)))

    dega, degb = _deg_call()(dstp, zeros_d, ones_e)
    u0 = _mm_call(xp, conv_W0)
    dega = dega.reshape(NP, 1)
    degb = degb.reshape(NP, 1)
    z = _scale_call(u0, dega, degb)

    for l in (0, 1):
        ya, yb = _scat_call()(z, srcp, dstp)
        _, b, g, be = layers[l]
        z = _mid_call(ya, yb, z, dega, degb, b, g, be, layers[l + 1][0])

    ya, yb = _scat_call()(z, srcp, dstp)
    _, b, g, be = layers[2]

    return _pool_call(ya, yb, z, dega, degb, b, g, be, bt,
                      scalar, scalar_W, vec(scalar_b),
                      cls_W1[:D], cls_W1[D:], vec(cls_b1),
                      cls_W2, vec(cls_b2))


# speedup vs baseline: 1.0041x; 1.0041x over previous
import functools
import math

import jax
import jax.numpy as jnp
from jax import lax
from jax.experimental import pallas as pl
from jax.experimental.pallas import tpu as pltpu
from jax.experimental.pallas import tpu_sc as plsc

N = 10000
E = 320000
G = 64
D = 128
NUM_CLASSES = 16
EPS = 1e-5
SCALE = 1.0 / math.sqrt(1.0 + EPS)

NP = 10240
BN = 2560
NGRID = NP // BN

EC = 128
NCH = 2560
NT = 16
NW = 2 * NT
CW = NCH // NW
RPT = NP // NT
HC = 40

_f32 = jnp.float32


@functools.cache
def _mesh():
    return plsc.VectorSubcoreMesh(core_axis_name="c", subcore_axis_name="s",
                                  num_cores=2, num_subcores=NT)


def _deg_body(dstp, zeros_d, ones_e, dega, degb, di_v, ones_v, deg_sh):
    c = lax.axis_index("c")
    s = lax.axis_index("s")
    w = c * NT + s
    pltpu.sync_copy(zeros_d, deg_sh.at[pl.ds(s * RPT, RPT)])
    pltpu.sync_copy(ones_e, ones_v)
    pltpu.sync_copy(dstp.at[pl.ds(w * CW, CW)], di_v)
    plsc.subcore_barrier()

    def body(j, carry):
        pltpu.sync_copy(ones_v, deg_sh.at[di_v.at[j]], add=True)
        return carry

    lax.fori_loop(0, CW, body, 0)
    plsc.subcore_barrier()

    @pl.when(c == 0)
    def _():
        pltpu.sync_copy(deg_sh.at[pl.ds(s * RPT, RPT)],
                        dega.at[pl.ds(s * RPT, RPT)])

    @pl.when(c == 1)
    def _():
        pltpu.sync_copy(deg_sh.at[pl.ds(s * RPT, RPT)],
                        degb.at[pl.ds(s * RPT, RPT)])


@functools.cache
def _deg_call():
    return pl.kernel(
        _deg_body,
        out_type=(jax.ShapeDtypeStruct((NP,), _f32),
                  jax.ShapeDtypeStruct((NP,), _f32)),
        mesh=_mesh(),
        scratch_types=[
            pltpu.VMEM((CW, EC), jnp.int32),
            pltpu.VMEM((EC,), _f32),
            pltpu.VMEM_SHARED((NP,), _f32),
        ],
    )


def _scat_body(z, srcp, dstp, ya, yb, si_v, di_v,
               rows_a, rows_b, gsa, gsb, y_sh):
    c = lax.axis_index("c")
    s = lax.axis_index("s")
    w = c * NT + s

    zv = jnp.zeros((16,), _f32)

    def zbody(i, carry):
        for kk in range(8):
            rows_a.at[i][pl.ds(kk * 16, 16)] = zv
        return carry

    lax.fori_loop(0, EC, zbody, 0)
    for r in range(RPT // EC):
        pltpu.sync_copy(rows_a, y_sh.at[pl.ds(s * RPT + r * EC, EC)])
    plsc.subcore_barrier()

    for h in range(CW // HC):
        base = w * CW + h * HC
        pltpu.sync_copy(srcp.at[pl.ds(base, HC)], si_v)
        pltpu.sync_copy(dstp.at[pl.ds(base, HC)], di_v)
        pltpu.async_copy(z.at[si_v.at[0]], rows_a, gsa)
        pltpu.async_copy(z.at[si_v.at[1]], rows_b, gsb)

        def body(k, carry):
            j0 = 2 * k
            pltpu.make_async_copy(z.at[si_v.at[0]], rows_a, gsa).wait()
            pltpu.sync_copy(rows_a, y_sh.at[di_v.at[j0]], add=True)
            pltpu.async_copy(z.at[si_v.at[j0 + 2]], rows_a, gsa)
            pltpu.make_async_copy(z.at[si_v.at[0]], rows_b, gsb).wait()
            pltpu.sync_copy(rows_b, y_sh.at[di_v.at[j0 + 1]], add=True)
            pltpu.async_copy(z.at[si_v.at[j0 + 3]], rows_b, gsb)
            return carry

        lax.fori_loop(0, HC // 2 - 1, body, 0)
        pltpu.make_async_copy(z.at[si_v.at[0]], rows_a, gsa).wait()
        pltpu.sync_copy(rows_a, y_sh.at[di_v.at[HC - 2]], add=True)
        pltpu.make_async_copy(z.at[si_v.at[0]], rows_b, gsb).wait()
        pltpu.sync_copy(rows_b, y_sh.at[di_v.at[HC - 1]], add=True)

    plsc.subcore_barrier()

    @pl.when(c == 0)
    def _():
        pltpu.sync_copy(y_sh.at[pl.ds(s * RPT, RPT)],
                        ya.at[pl.ds(s * RPT, RPT)])

    @pl.when(c == 1)
    def _():
        pltpu.sync_copy(y_sh.at[pl.ds(s * RPT, RPT)],
                        yb.at[pl.ds(s * RPT, RPT)])


@functools.cache
def _scat_call():
    return pl.kernel(
        _scat_body,
        out_type=(jax.ShapeDtypeStruct((NP, D), _f32),
                  jax.ShapeDtypeStruct((NP, D), _f32)),
        mesh=_mesh(),
        scratch_types=[
            pltpu.VMEM((HC, EC), jnp.int32),
            pltpu.VMEM((HC, EC), jnp.int32),
            pltpu.VMEM((EC, D), _f32),
            pltpu.VMEM((EC, D), _f32),
            pltpu.SemaphoreType.DMA,
            pltpu.SemaphoreType.DMA,
            pltpu.VMEM_SHARED((NP, D), _f32),
        ],
    )


def _dis(dega, degb):
    return lax.rsqrt(1.0 + dega[...] + degb[...])


def _z0_body(x_ref, dega, degb, w_ref, z_ref):
    z_ref[...] = _dis(dega, degb) * jnp.dot(
        x_ref[...], w_ref[...], preferred_element_type=_f32)


_row_spec = pl.BlockSpec((BN, D), lambda i: (i, 0))
_deg_spec = pl.BlockSpec((BN, 1), lambda i: (i, 0))
_vec_spec = pl.BlockSpec((1, D), lambda i: (0, 0))
_w_spec = pl.BlockSpec((D, D), lambda i: (0, 0))

_z0_call = pl.pallas_call(
    _z0_body,
    grid=(NGRID,),
    in_specs=[_row_spec, _deg_spec, _deg_spec, _w_spec],
    out_specs=_row_spec,
    out_shape=jax.ShapeDtypeStruct((NP, D), _f32),
)


def _h_block(ya, yb, z, dega, degb, bp, g, be):
    dis = _dis(dega, degb)
    h = (dis * (ya[...] + yb[...] + z[...]) + bp[...]) * (g[...] * SCALE)
    return dis, jnp.maximum(h + be[...], 0.0)


def _mid_body(ya, yb, z, dega, degb, bp, g, be, w_ref, zo):
    dis, h = _h_block(ya, yb, z, dega, degb, bp, g, be)
    zo[...] = dis * jnp.dot(h, w_ref[...], preferred_element_type=_f32)


_mid_call = pl.pallas_call(
    _mid_body,
    grid=(NGRID,),
    in_specs=[_row_spec] * 3 + [_deg_spec] * 2 + [_vec_spec] * 3 + [_w_spec],
    out_specs=_row_spec,
    out_shape=jax.ShapeDtypeStruct((NP, D), _f32),
)


def _pool_body(ya, yb, z, dega, degb, bp, g, be, bt,
               sc, sw, sb, w1g, w1s, b1, w2, b2,
               out_ref, p_acc, cnt_acc):
    i = pl.program_id(0)
    _, h = _h_block(ya, yb, z, dega, degb, bp, g, be)
    gid = lax.broadcasted_iota(jnp.int32, (1, G), 1)
    mask = (bt[...] == gid).astype(_f32)
    dn = (((0,), (0,)), ((), ()))
    p = lax.dot_general(mask, h, dn, preferred_element_type=_f32)
    cnt = lax.dot_general(mask, jnp.ones((BN, 1), _f32), dn,
                          preferred_element_type=_f32)

    @pl.when(i == 0)
    def _():
        p_acc[...] = jnp.zeros_like(p_acc)
        cnt_acc[...] = jnp.zeros_like(cnt_acc)

    p_acc[...] += p
    cnt_acc[...] += cnt

    @pl.when(i == NGRID - 1)
    def _():
        ge = p_acc[...] / jnp.maximum(cnt_acc[...], 1.0)
        se = jnp.maximum(jnp.dot(sc[...], sw[...],
                                 preferred_element_type=_f32) + sb[...], 0.0)
        hc = jnp.maximum(
            jnp.dot(ge, w1g[...], preferred_element_type=_f32)
            + jnp.dot(se, w1s[...], preferred_element_type=_f32)
            + b1[...], 0.0)
        out_ref[...] = jnp.dot(hc, w2[...],
                               preferred_element_type=_f32) + b2[...]


def _const_spec(r, c):
    return pl.BlockSpec((r, c), lambda i: (0, 0))


_pool_call = pl.pallas_call(
    _pool_body,
    grid=(NGRID,),
    in_specs=[_row_spec] * 3 + [_deg_spec] * 2 + [_vec_spec] * 3
             + [pl.BlockSpec((BN, 1), lambda i: (i, 0)),
                _const_spec(G, 8), _const_spec(8, D // 2),
                _const_spec(1, D // 2), _const_spec(D, D),
                _const_spec(D // 2, D), _const_spec(1, D),
                _const_spec(D, NUM_CLASSES), _const_spec(1, NUM_CLASSES)],
    out_specs=_const_spec(G, NUM_CLASSES),
    out_shape=jax.ShapeDtypeStruct((G, NUM_CLASSES), _f32),
    scratch_shapes=[pltpu.VMEM((G, D), _f32), pltpu.VMEM((G, 1), _f32)],
)


def kernel(x, edge_index, batch, scalar,
           conv_W0, conv_b0, conv_W1, conv_b1, conv_W2, conv_b2,
           bn_g0, bn_b0, bn_g1, bn_b1, bn_g2, bn_b2,
           scalar_W, scalar_b, cls_W1, cls_b1, cls_W2, cls_b2):
    npad = NCH * EC - E
    pads = N + (jnp.arange(npad, dtype=jnp.int32) % (NP - N))
    srcp = jnp.concatenate([edge_index[0], pads]).reshape(NCH, EC)
    dstp = jnp.concatenate([edge_index[1], pads]).reshape(NCH, EC)

    xp = jnp.pad(x, ((0, NP - N), (0, 0)))
    bt = jnp.concatenate(
        [batch, jnp.full((NP - N,), G, jnp.int32)]).reshape(NP, 1)

    zeros_d = jnp.zeros((RPT,), _f32)
    ones_e = jnp.ones((EC,), _f32)

    def vec(v):
        return v.reshape(1, v.shape[0])

    layers = ((conv_W0, vec(conv_b0), vec(bn_g0), vec(bn_b0)),
              (conv_W1, vec(conv_b1), vec(bn_g1), vec(bn_b1)),
              (conv_W2, vec(conv_b2), vec(bn_g2), vec(bn_b2)))

    dega, degb = _deg_call()(dstp, zeros_d, ones_e)
    dega = dega.reshape(NP, 1)
    degb = degb.reshape(NP, 1)

    z = _z0_call(xp, dega, degb, conv_W0)

    for l in (0, 1):
        ya, yb = _scat_call()(z, srcp, dstp)
        _, b, g, be = layers[l]
        z = _mid_call(ya, yb, z, dega, degb, b, g, be, layers[l + 1][0])

    ya, yb = _scat_call()(z, srcp, dstp)
    _, b, g, be = layers[2]

    return _pool_call(ya, yb, z, dega, degb, b, g, be, bt,
                      scalar, scalar_W, vec(scalar_b),
                      cls_W1[:D], cls_W1[D:], vec(cls_b1),
                      cls_W2, vec(cls_b2))
